# baseline (speedup 1.0000x reference)
import functools

import jax
import jax.numpy as jnp
from jax import lax
from jax.experimental import pallas as pl
from jax.experimental.pallas import tpu as pltpu

L = 512
D = 128
NT = 2 * 32 + 1
SPAD = 1032


def _tc_body(table_ref, out_ref, s8_ref):
    i = pl.program_id(0)

    @pl.when(i == 0)
    def _():
        for r in range(8):
            u = lax.broadcasted_iota(jnp.int32, (SPAD, NT), 0) + r
            v = lax.broadcasted_iota(jnp.int32, (SPAD, NT), 1)
            m = (L - 1) - u
            g = jnp.where(m >= 0, 32,
                          jnp.where(m >= -32, m + 32,
                                    jnp.where(m >= -64, m + 97, 33)))
            onehot = (v == g).astype(jnp.float32)
            s8_ref[r] = jnp.dot(onehot, table_ref[...],
                                preferred_element_type=jnp.float32)

    for rr in range(32):
        o = (L - 1) - (32 * i + rr)
        r = lax.rem(o, 8)
        a = pl.multiple_of(o - r, 8)
        out_ref[rr] = s8_ref[r, pl.ds(a, L), :]


_tc_call = pl.pallas_call(
    _tc_body,
    grid=(L // 32,),
    in_specs=[pl.BlockSpec((NT, D), lambda i: (0, 0))],
    out_specs=pl.BlockSpec((32, L, D), lambda i: (i, 0, 0)),
    out_shape=jax.ShapeDtypeStruct((L, L, D), jnp.float32),
    scratch_shapes=[pltpu.VMEM((8, SPAD, D), jnp.float32)],
)


@jax.jit
def kernel(idx, pos_embedding):
    del idx
    return _tc_call(pos_embedding)
